# baseline (speedup 1.0000x reference)
import jax
import jax.numpy as jnp
from jax.experimental import pallas as pl
from jax.experimental.pallas import tpu as pltpu

_E = 64
_K = 8
_BLOCK = 1024


_SPLIT = 1


def _router_block(*refs):
    x_refs = refs[:_SPLIT]
    w_ref, logits_ref, weights_ref, experts_ref = refs[_SPLIT:]
    dk = w_ref.shape[0] // _SPLIT
    logits = None
    for j in range(_SPLIT):
        x = x_refs[j][...].astype(jnp.bfloat16)
        w = w_ref[j * dk:(j + 1) * dk, :].astype(jnp.bfloat16)
        part = jax.lax.dot_general(
            x, w, (((1,), (0,)), ((), ())), preferred_element_type=jnp.float32
        )
        logits = part if logits is None else logits + part

    t = logits.shape[0]
    lt = logits.T
    logits_ref[...] = lt
    _C = 128
    iota_f = jax.lax.broadcasted_iota(jnp.int32, (_E, _C), 0).astype(jnp.float32)
    wt_cols, et_cols = [], []
    for c in range(0):
        work = lt[:, c * _C:(c + 1) * _C]
        vals, idxs = [], []
        for _ in range(_K):
            m = jnp.max(work, axis=0, keepdims=True)
            idx = jnp.min(jnp.where(work == m, iota_f, float(_E)),
                          axis=0, keepdims=True)
            vals.append(m)
            idxs.append(idx)
            work = jnp.where(iota_f == idx, -jnp.inf, work)
        g = jnp.concatenate(vals, axis=0)
        e = jnp.concatenate(idxs, axis=0)
        ew = jnp.exp(g - g[0:1, :])
        wt_cols.append(ew / jnp.sum(ew, axis=0, keepdims=True))
        et_cols.append(e)
    weights_ref[...] = lt[:_K, :]
    experts_ref[...] = lt[:_K, :].astype(jnp.int32)


def kernel(hidden_states, W):
    b, s, d = hidden_states.shape
    n = b * s
    x = hidden_states.reshape(n, d)
    grid = (n // _BLOCK,)
    dk = d // _SPLIT
    logits, weights, experts = pl.pallas_call(
        _router_block,
        grid=grid,
        in_specs=[
            pl.BlockSpec((_BLOCK, dk), lambda i, j=j: (i, j))
            for j in range(_SPLIT)
        ] + [
            pl.BlockSpec((d, _E), lambda i: (0, 0)),
        ],
        out_specs=[
            pl.BlockSpec((_E, _BLOCK), lambda i: (0, i)),
            pl.BlockSpec((_K, _BLOCK), lambda i: (0, i)),
            pl.BlockSpec((_K, _BLOCK), lambda i: (0, i)),
        ],
        out_shape=[
            jax.ShapeDtypeStruct((_E, n), jnp.float32),
            jax.ShapeDtypeStruct((_K, n), jnp.float32),
            jax.ShapeDtypeStruct((_K, n), jnp.int32),
        ],
        compiler_params=pltpu.CompilerParams(
            dimension_semantics=("arbitrary",),
            vmem_limit_bytes=40 * 1024 * 1024,
        ),
    )(*([x] * _SPLIT), W)
    return (weights.T, experts.T, logits.T)


# speedup vs baseline: 1.0014x; 1.0014x over previous
import jax
import jax.numpy as jnp
from jax.experimental import pallas as pl
from jax.experimental.pallas import tpu as pltpu

_E = 64
_K = 8
_BLOCK = 1024


_SPLIT = 1


def _router_block(*refs):
    x_refs = refs[:_SPLIT]
    w_ref, logits_ref, weights_ref, experts_ref = refs[_SPLIT:]
    dk = w_ref.shape[0] // _SPLIT
    logits = None
    for j in range(_SPLIT):
        x = x_refs[j][...].astype(jnp.bfloat16)
        w = w_ref[j * dk:(j + 1) * dk, :].astype(jnp.bfloat16)
        part = jax.lax.dot_general(
            x, w, (((1,), (0,)), ((), ())), preferred_element_type=jnp.float32
        )
        logits = part if logits is None else logits + part

    t = logits.shape[0]
    lt = logits.T
    logits_ref[...] = lt
    _C = 128
    iota_f = jax.lax.broadcasted_iota(jnp.int32, (_E, _C), 0).astype(jnp.float32)
    wt_cols, et_cols = [], []
    for c in range(t // _C):
        work = lt[:, c * _C:(c + 1) * _C]
        vals, idxs = [], []
        for _ in range(_K):
            m = jnp.max(work, axis=0, keepdims=True)
            idx = jnp.min(jnp.where(work == m, iota_f, float(_E)),
                          axis=0, keepdims=True)
            vals.append(m)
            idxs.append(idx)
            work = jnp.where(iota_f == idx, -jnp.inf, work)
        g = jnp.concatenate(vals, axis=0)
        e = jnp.concatenate(idxs, axis=0)
        ew = jnp.exp(g - g[0:1, :])
        wt_cols.append(ew / jnp.sum(ew, axis=0, keepdims=True))
        et_cols.append(e)
    w_t = jnp.concatenate(wt_cols, axis=1)
    e_t = jnp.concatenate(et_cols, axis=1)
    weights_ref[...] = w_t
    experts_ref[...] = e_t.astype(jnp.int32)


def kernel(hidden_states, W):
    b, s, d = hidden_states.shape
    n = b * s
    x = hidden_states.reshape(n, d)
    grid = (n // _BLOCK,)
    dk = d // _SPLIT
    logits, weights, experts = pl.pallas_call(
        _router_block,
        grid=grid,
        in_specs=[
            pl.BlockSpec((_BLOCK, dk), lambda i, j=j: (i, j))
            for j in range(_SPLIT)
        ] + [
            pl.BlockSpec((d, _E), lambda i: (0, 0)),
        ],
        out_specs=[
            pl.BlockSpec((_E, _BLOCK), lambda i: (0, i)),
            pl.BlockSpec((_K, _BLOCK), lambda i: (0, i)),
            pl.BlockSpec((_K, _BLOCK), lambda i: (0, i)),
        ],
        out_shape=[
            jax.ShapeDtypeStruct((_E, n), jnp.float32),
            jax.ShapeDtypeStruct((_K, n), jnp.float32),
            jax.ShapeDtypeStruct((_K, n), jnp.int32),
        ],
        compiler_params=pltpu.CompilerParams(
            dimension_semantics=("arbitrary",),
            vmem_limit_bytes=40 * 1024 * 1024,
        ),
    )(*([x] * _SPLIT), W)
    return (weights.T, experts.T, logits.T)
